# baseline (speedup 1.0000x reference)
import functools

import numpy as np

import jax
import jax.numpy as jnp
from jax import lax
from jax.experimental import pallas as pl
from jax.experimental.pallas import tpu as pltpu
from jax.experimental.pallas import tpu_sc as plsc

_B = 125


def _r8(d):
    return 4


@functools.lru_cache(maxsize=None)
def _sc_scatter_fn(n_ext, d, nw, ns, nc, g, const_rows=False):
    r8 = _r8(d)
    t = g // r8
    c = r8 * _B
    nbuf = 1 if const_rows else 2
    rpt = n_ext // ns
    mesh = plsc.VectorSubcoreMesh(core_axis_name="c", subcore_axis_name="s")

    @functools.partial(
        pl.kernel,
        out_type=jax.ShapeDtypeStruct((nc, n_ext, d), jnp.float32),
        mesh=mesh,
        scratch_types=[
            pltpu.VMEM((t, c), jnp.int32),
            pltpu.VMEM((t, c), jnp.int32),
            pltpu.VMEM((nbuf, c, d), jnp.float32),
            pltpu.VMEM_SHARED((n_ext, d), jnp.float32),
            pltpu.SemaphoreType.DMA((2,)),
            pltpu.SemaphoreType.DMA((2,)),
            pltpu.SemaphoreType.DMA,
        ],
        compiler_params=pltpu.CompilerParams(use_tc_tiling_on_sc=False),
    )
    def scat(u_hbm, src_hbm, dst_hbm, out_hbm, src_v, dst_v, rows_v, acc_sh,
             gsem, ssem, isem):
        cid = lax.axis_index("c")
        sid = lax.axis_index("s")
        wid = sid * nc + cid
        r0 = sid * rpt
        pltpu.sync_copy(src_hbm.at[wid], src_v)
        pltpu.sync_copy(dst_hbm.at[wid], dst_v)
        pltpu.sync_copy(u_hbm.at[pl.ds(r0, rpt)], acc_sh.at[pl.ds(r0, rpt)])
        gd = {}
        for k in range(min(nbuf, t)):
            gd[k] = pltpu.async_copy(
                u_hbm.at[src_v.at[k]], rows_v.at[k % nbuf], gsem.at[k % nbuf])
        plsc.subcore_barrier()
        if const_rows:
            gd[0].wait()
            for k in range(t):
                pltpu.sync_copy(rows_v.at[0], acc_sh.at[dst_v.at[k]], add=True)
        else:
            for k in range(t):
                b = k % nbuf
                gd.pop(k).wait()
                pltpu.sync_copy(rows_v.at[b], acc_sh.at[dst_v.at[k]], add=True)
                if k + nbuf < t:
                    gd[k + nbuf] = pltpu.async_copy(
                        u_hbm.at[src_v.at[k + nbuf]], rows_v.at[b], gsem.at[b])
        plsc.subcore_barrier()
        pltpu.sync_copy(acc_sh.at[pl.ds(r0, rpt)], out_hbm.at[cid, pl.ds(r0, rpt)])

    return scat


def _tc_stage1(x, W1, degp, n, n_ext):
    d1 = W1.shape[1]

    def body(x_ref, w_ref, degp_ref, u_ref, dinv_ref):
        deg = degp_ref[0, :n, :1] + degp_ref[1, :n, :1] - 1.0
        dinv = lax.rsqrt(deg)
        dinv_ref[...] = dinv
        hw = jnp.dot(x_ref[...], w_ref[...], preferred_element_type=jnp.float32)
        u_ref[...] = jnp.pad(hw * dinv, ((0, n_ext - n), (0, 0)))

    return pl.pallas_call(
        body,
        out_shape=(jax.ShapeDtypeStruct((n_ext, d1), jnp.float32),
                   jax.ShapeDtypeStruct((n, 1), jnp.float32)),
    )(x, W1, degp)


def _tc_mid(sp, u, dinv, b, gamma, beta, W, n, n_ext):
    dn = W.shape[1]

    def body(sp_ref, u_ref, dinv_ref, b_ref, g_ref, be_ref, w_ref, out_ref):
        s = sp_ref[0, :n] + sp_ref[1, :n] - u_ref[:n]
        agg = s * dinv_ref[...] + b_ref[...]
        m = jnp.mean(agg, axis=0, keepdims=True)
        v = jnp.mean((agg - m) ** 2, axis=0, keepdims=True)
        h = (agg - m) * lax.rsqrt(v + 1e-5) * g_ref[...] + be_ref[...]
        h = jnp.maximum(h, 0.0)
        un = (jnp.dot(h, w_ref[...], preferred_element_type=jnp.float32)
              * dinv_ref[...])
        out_ref[...] = jnp.pad(un, ((0, n_ext - n), (0, 0)))

    return pl.pallas_call(
        body, out_shape=jax.ShapeDtypeStruct((n_ext, dn), jnp.float32)
    )(sp, u, dinv, b.reshape(1, -1), gamma.reshape(1, -1), beta.reshape(1, -1), W)


def _tc_final(sp, u, dinv, b, n):
    def body(sp_ref, u_ref, dinv_ref, b_ref, out_ref):
        s = sp_ref[0, :n] + sp_ref[1, :n] - u_ref[:n]
        out_ref[...] = s * dinv_ref[...] + b_ref[...]

    return pl.pallas_call(
        body, out_shape=jax.ShapeDtypeStruct((n, u.shape[1]), jnp.float32)
    )(sp, u, dinv, b.reshape(1, -1))


def kernel(x, W1, b1, gamma1, beta1, W2, b2, gamma2, beta2, W3, b3, edge_index):
    n = x.shape[0]
    e = edge_index.shape[1]
    info = plsc.get_sparse_core_info()
    nc, ns = info.num_cores, info.num_subcores
    nw = nc * ns

    blk = nw * _r8(0) * _B
    e_pad = -(-e // blk) * blk
    src = edge_index[0]
    dst = edge_index[1]
    if e_pad != e:
        fill = jnp.full((e_pad - e,), n, dtype=jnp.int32)
        src = jnp.concatenate([src, fill])
        dst = jnp.concatenate([dst, fill])
    g = e_pad // (nw * _B)
    src_r = src.reshape(nw, g, _B)
    dst_r = dst.reshape(nw, g, _B)
    n_ext = -(-(n + 1) // 128) * 128

    def shaped(a, r8):
        return a.reshape(nw, g // r8, r8 * _B)

    ones_np = np.zeros((n_ext, 16), np.float32)
    ones_np[:n] = 1.0
    ones_tab = jnp.asarray(ones_np)
    degp = _sc_scatter_fn(n_ext, 16, nw, ns, nc, g, const_rows=True)(
        ones_tab, shaped(src_r, _r8(16)), shaped(dst_r, _r8(16)))
    u1, dinv = _tc_stage1(x, W1, degp, n, n_ext)

    d1, d2, d3 = W1.shape[1], W2.shape[1], W3.shape[1]
    sp1 = _sc_scatter_fn(n_ext, d1, nw, ns, nc, g)(
        u1, shaped(src_r, _r8(d1)), shaped(dst_r, _r8(d1)))
    u2 = _tc_mid(sp1, u1, dinv, b1, gamma1, beta1, W2, n, n_ext)

    sp2 = _sc_scatter_fn(n_ext, d2, nw, ns, nc, g)(
        u2, shaped(src_r, _r8(d2)), shaped(dst_r, _r8(d2)))
    u3 = _tc_mid(sp2, u2, dinv, b2, gamma2, beta2, W3, n, n_ext)

    sp3 = _sc_scatter_fn(n_ext, d3, nw, ns, nc, g)(
        u3, shaped(src_r, _r8(d3)), shaped(dst_r, _r8(d3)))
    return _tc_final(sp3, u3, dinv, b3, n)


# speedup vs baseline: 1.0195x; 1.0195x over previous
import functools

import numpy as np

import jax
import jax.numpy as jnp
from jax import lax
from jax.experimental import pallas as pl
from jax.experimental.pallas import tpu as pltpu
from jax.experimental.pallas import tpu_sc as plsc

_B = 125


def _r8(d):
    return 8 if d <= 32 else 4


@functools.lru_cache(maxsize=None)
def _sc_scatter_fn(n_ext, d, nw, ns, nc, g, const_rows=False):
    r8 = _r8(d)
    t = g // r8
    c = r8 * _B
    nbuf = 1 if const_rows else 2
    rpt = n_ext // ns
    mesh = plsc.VectorSubcoreMesh(core_axis_name="c", subcore_axis_name="s")

    @functools.partial(
        pl.kernel,
        out_type=jax.ShapeDtypeStruct((nc, n_ext, d), jnp.float32),
        mesh=mesh,
        scratch_types=[
            pltpu.VMEM((t, c), jnp.int32),
            pltpu.VMEM((t, c), jnp.int32),
            pltpu.VMEM((nbuf, c, d), jnp.float32),
            pltpu.VMEM_SHARED((n_ext, d), jnp.float32),
            pltpu.SemaphoreType.DMA((2,)),
            pltpu.SemaphoreType.DMA((2,)),
            pltpu.SemaphoreType.DMA,
        ],
        compiler_params=pltpu.CompilerParams(use_tc_tiling_on_sc=False),
    )
    def scat(u_hbm, src_hbm, dst_hbm, out_hbm, src_v, dst_v, rows_v, acc_sh,
             gsem, ssem, isem):
        cid = lax.axis_index("c")
        sid = lax.axis_index("s")
        wid = sid * nc + cid
        r0 = sid * rpt
        pltpu.sync_copy(src_hbm.at[wid], src_v)
        pltpu.sync_copy(dst_hbm.at[wid], dst_v)
        pltpu.sync_copy(u_hbm.at[pl.ds(r0, rpt)], acc_sh.at[pl.ds(r0, rpt)])
        gd = {}
        for k in range(min(nbuf, t)):
            gd[k] = pltpu.async_copy(
                u_hbm.at[src_v.at[k]], rows_v.at[k % nbuf], gsem.at[k % nbuf])
        plsc.subcore_barrier()
        if const_rows:
            gd[0].wait()
            for k in range(t):
                pltpu.sync_copy(rows_v.at[0], acc_sh.at[dst_v.at[k]], add=True)
        else:
            for k in range(t):
                b = k % nbuf
                gd.pop(k).wait()
                pltpu.sync_copy(rows_v.at[b], acc_sh.at[dst_v.at[k]], add=True)
                if k + nbuf < t:
                    gd[k + nbuf] = pltpu.async_copy(
                        u_hbm.at[src_v.at[k + nbuf]], rows_v.at[b], gsem.at[b])
        plsc.subcore_barrier()
        pltpu.sync_copy(acc_sh.at[pl.ds(r0, rpt)], out_hbm.at[cid, pl.ds(r0, rpt)])

    return scat


def _tc_stage1(x, W1, degp, n, n_ext):
    d1 = W1.shape[1]

    def body(x_ref, w_ref, degp_ref, u_ref, dinv_ref):
        deg = degp_ref[0, :n, :1] + degp_ref[1, :n, :1] - 1.0
        dinv = lax.rsqrt(deg)
        dinv_ref[...] = dinv
        hw = jnp.dot(x_ref[...], w_ref[...], preferred_element_type=jnp.float32)
        u_ref[...] = jnp.pad(hw * dinv, ((0, n_ext - n), (0, 0)))

    return pl.pallas_call(
        body,
        out_shape=(jax.ShapeDtypeStruct((n_ext, d1), jnp.float32),
                   jax.ShapeDtypeStruct((n, 1), jnp.float32)),
    )(x, W1, degp)


def _tc_mid(sp, u, dinv, b, gamma, beta, W, n, n_ext):
    dn = W.shape[1]

    def body(sp_ref, u_ref, dinv_ref, b_ref, g_ref, be_ref, w_ref, out_ref):
        s = sp_ref[0, :n] + sp_ref[1, :n] - u_ref[:n]
        agg = s * dinv_ref[...] + b_ref[...]
        m = jnp.mean(agg, axis=0, keepdims=True)
        v = jnp.mean((agg - m) ** 2, axis=0, keepdims=True)
        h = (agg - m) * lax.rsqrt(v + 1e-5) * g_ref[...] + be_ref[...]
        h = jnp.maximum(h, 0.0)
        un = (jnp.dot(h, w_ref[...], preferred_element_type=jnp.float32)
              * dinv_ref[...])
        out_ref[...] = jnp.pad(un, ((0, n_ext - n), (0, 0)))

    return pl.pallas_call(
        body, out_shape=jax.ShapeDtypeStruct((n_ext, dn), jnp.float32)
    )(sp, u, dinv, b.reshape(1, -1), gamma.reshape(1, -1), beta.reshape(1, -1), W)


def _tc_final(sp, u, dinv, b, n):
    def body(sp_ref, u_ref, dinv_ref, b_ref, out_ref):
        s = sp_ref[0, :n] + sp_ref[1, :n] - u_ref[:n]
        out_ref[...] = s * dinv_ref[...] + b_ref[...]

    return pl.pallas_call(
        body, out_shape=jax.ShapeDtypeStruct((n, u.shape[1]), jnp.float32)
    )(sp, u, dinv, b.reshape(1, -1))


def kernel(x, W1, b1, gamma1, beta1, W2, b2, gamma2, beta2, W3, b3, edge_index):
    n = x.shape[0]
    e = edge_index.shape[1]
    info = plsc.get_sparse_core_info()
    nc, ns = info.num_cores, info.num_subcores
    nw = nc * ns

    blk = nw * 8 * _B
    e_pad = -(-e // blk) * blk
    src = edge_index[0]
    dst = edge_index[1]
    if e_pad != e:
        fill = jnp.full((e_pad - e,), n, dtype=jnp.int32)
        src = jnp.concatenate([src, fill])
        dst = jnp.concatenate([dst, fill])
    g = e_pad // (nw * _B)
    src_r = src.reshape(nw, g, _B)
    dst_r = dst.reshape(nw, g, _B)
    n_ext = -(-(n + 1) // 128) * 128

    def shaped(a, r8):
        return a.reshape(nw, g // r8, r8 * _B)

    ones_np = np.zeros((n_ext, 16), np.float32)
    ones_np[:n] = 1.0
    ones_tab = jnp.asarray(ones_np)
    degp = _sc_scatter_fn(n_ext, 16, nw, ns, nc, g, const_rows=True)(
        ones_tab, shaped(src_r, _r8(16)), shaped(dst_r, _r8(16)))
    u1, dinv = _tc_stage1(x, W1, degp, n, n_ext)

    d1, d2, d3 = W1.shape[1], W2.shape[1], W3.shape[1]
    sp1 = _sc_scatter_fn(n_ext, d1, nw, ns, nc, g)(
        u1, shaped(src_r, _r8(d1)), shaped(dst_r, _r8(d1)))
    u2 = _tc_mid(sp1, u1, dinv, b1, gamma1, beta1, W2, n, n_ext)

    sp2 = _sc_scatter_fn(n_ext, d2, nw, ns, nc, g)(
        u2, shaped(src_r, _r8(d2)), shaped(dst_r, _r8(d2)))
    u3 = _tc_mid(sp2, u2, dinv, b2, gamma2, beta2, W3, n, n_ext)

    sp3 = _sc_scatter_fn(n_ext, d3, nw, ns, nc, g)(
        u3, shaped(src_r, _r8(d3)), shaped(dst_r, _r8(d3)))
    return _tc_final(sp3, u3, dinv, b3, n)


# speedup vs baseline: 1.1315x; 1.1099x over previous
import functools

import numpy as np

import jax
import jax.numpy as jnp
from jax import lax
from jax.experimental import pallas as pl
from jax.experimental.pallas import tpu as pltpu
from jax.experimental.pallas import tpu_sc as plsc

_B = 125


def _r8(d):
    return 8 if d <= 32 else 4


@functools.lru_cache(maxsize=None)
def _sc_scatter_fn(n_ext, d, nw, ns, nc, g, const_rows=False):
    r8 = _r8(d)
    t = g // r8
    c = r8 * _B
    nbuf = 1 if const_rows else 2
    rpt = n_ext // ns
    mesh = plsc.VectorSubcoreMesh(core_axis_name="c", subcore_axis_name="s")

    @functools.partial(
        pl.kernel,
        out_type=jax.ShapeDtypeStruct((nc, n_ext, 128), jnp.float32),
        mesh=mesh,
        scratch_types=[
            pltpu.VMEM((t, c), jnp.int32),
            pltpu.VMEM((t, c), jnp.int32),
            pltpu.VMEM((nbuf, c, d), jnp.float32),
            pltpu.VMEM_SHARED((n_ext, d), jnp.float32),
            pltpu.SemaphoreType.DMA((2,)),
            pltpu.SemaphoreType.DMA((2,)),
            pltpu.SemaphoreType.DMA,
        ],
        compiler_params=pltpu.CompilerParams(use_tc_tiling_on_sc=False),
    )
    def scat(u_hbm, src_hbm, dst_hbm, out_hbm, src_v, dst_v, rows_v, acc_sh,
             gsem, ssem, isem):
        cid = lax.axis_index("c")
        sid = lax.axis_index("s")
        wid = sid * nc + cid
        r0 = sid * rpt
        pltpu.sync_copy(src_hbm.at[wid], src_v)
        pltpu.sync_copy(dst_hbm.at[wid], dst_v)
        pltpu.sync_copy(u_hbm.at[pl.ds(r0, rpt)], acc_sh.at[pl.ds(r0, rpt)])
        gd = {}
        for k in range(min(nbuf, t)):
            gd[k] = pltpu.async_copy(
                u_hbm.at[src_v.at[k]], rows_v.at[k % nbuf], gsem.at[k % nbuf])
        plsc.subcore_barrier()
        if const_rows:
            gd[0].wait()
            for k in range(t):
                pltpu.sync_copy(rows_v.at[0], acc_sh.at[dst_v.at[k]], add=True)
        else:
            for k in range(t):
                b = k % nbuf
                gd.pop(k).wait()
                pltpu.sync_copy(rows_v.at[b], acc_sh.at[dst_v.at[k]], add=True)
                if k + nbuf < t:
                    gd[k + nbuf] = pltpu.async_copy(
                        u_hbm.at[src_v.at[k + nbuf]], rows_v.at[b], gsem.at[b])
        plsc.subcore_barrier()
        pltpu.sync_copy(acc_sh.at[pl.ds(r0, rpt)],
                        out_hbm.at[cid, pl.ds(r0, rpt), pl.ds(0, d)])

    return scat


def _tc_stage1(x, W1, degp, n, n_ext):
    d1 = W1.shape[1]

    def body(x_ref, w_ref, degp_ref, u_ref, dinv_ref):
        deg = degp_ref[0, :n, :1] + degp_ref[1, :n, :1] - 1.0
        dinv = lax.rsqrt(deg)
        dinv_ref[...] = dinv
        hw = jnp.dot(x_ref[...], w_ref[...], preferred_element_type=jnp.float32)
        u_ref[...] = jnp.pad(hw * dinv, ((0, n_ext - n), (0, 0)))

    return pl.pallas_call(
        body,
        out_shape=(jax.ShapeDtypeStruct((n_ext, d1), jnp.float32),
                   jax.ShapeDtypeStruct((n, 1), jnp.float32)),
    )(x, W1, degp)


def _tc_mid(sp, u, dinv, b, gamma, beta, W, n, n_ext):
    dn = W.shape[1]

    d = u.shape[1]

    def body(sp_ref, u_ref, dinv_ref, b_ref, g_ref, be_ref, w_ref, out_ref):
        s = sp_ref[0, :n, :d] + sp_ref[1, :n, :d] - u_ref[:n]
        agg = s * dinv_ref[...] + b_ref[...]
        m = jnp.mean(agg, axis=0, keepdims=True)
        v = jnp.mean((agg - m) ** 2, axis=0, keepdims=True)
        h = (agg - m) * lax.rsqrt(v + 1e-5) * g_ref[...] + be_ref[...]
        h = jnp.maximum(h, 0.0)
        un = (jnp.dot(h, w_ref[...], preferred_element_type=jnp.float32)
              * dinv_ref[...])
        out_ref[...] = jnp.pad(un, ((0, n_ext - n), (0, 0)))

    return pl.pallas_call(
        body, out_shape=jax.ShapeDtypeStruct((n_ext, dn), jnp.float32)
    )(sp, u, dinv, b.reshape(1, -1), gamma.reshape(1, -1), beta.reshape(1, -1), W)


def _tc_final(sp, u, dinv, b, n):
    d = u.shape[1]

    def body(sp_ref, u_ref, dinv_ref, b_ref, out_ref):
        s = sp_ref[0, :n, :d] + sp_ref[1, :n, :d] - u_ref[:n]
        out_ref[...] = s * dinv_ref[...] + b_ref[...]

    return pl.pallas_call(
        body, out_shape=jax.ShapeDtypeStruct((n, u.shape[1]), jnp.float32)
    )(sp, u, dinv, b.reshape(1, -1))


def kernel(x, W1, b1, gamma1, beta1, W2, b2, gamma2, beta2, W3, b3, edge_index):
    n = x.shape[0]
    e = edge_index.shape[1]
    info = plsc.get_sparse_core_info()
    nc, ns = info.num_cores, info.num_subcores
    nw = nc * ns

    blk = nw * 8 * _B
    e_pad = -(-e // blk) * blk
    src = edge_index[0]
    dst = edge_index[1]
    if e_pad != e:
        fill = jnp.full((e_pad - e,), n, dtype=jnp.int32)
        src = jnp.concatenate([src, fill])
        dst = jnp.concatenate([dst, fill])
    g = e_pad // (nw * _B)
    src_r = src.reshape(nw, g, _B)
    dst_r = dst.reshape(nw, g, _B)
    n_ext = -(-(n + 1) // 128) * 128

    def shaped(a, r8):
        return a.reshape(nw, g // r8, r8 * _B)

    ones_np = np.zeros((n_ext, 16), np.float32)
    ones_np[:n] = 1.0
    ones_tab = jnp.asarray(ones_np)
    degp = _sc_scatter_fn(n_ext, 16, nw, ns, nc, g, const_rows=True)(
        ones_tab, shaped(src_r, _r8(16)), shaped(dst_r, _r8(16)))
    u1, dinv = _tc_stage1(x, W1, degp, n, n_ext)

    d1, d2, d3 = W1.shape[1], W2.shape[1], W3.shape[1]
    sp1 = _sc_scatter_fn(n_ext, d1, nw, ns, nc, g)(
        u1, shaped(src_r, _r8(d1)), shaped(dst_r, _r8(d1)))
    u2 = _tc_mid(sp1, u1, dinv, b1, gamma1, beta1, W2, n, n_ext)

    sp2 = _sc_scatter_fn(n_ext, d2, nw, ns, nc, g)(
        u2, shaped(src_r, _r8(d2)), shaped(dst_r, _r8(d2)))
    u3 = _tc_mid(sp2, u2, dinv, b2, gamma2, beta2, W3, n, n_ext)

    sp3 = _sc_scatter_fn(n_ext, d3, nw, ns, nc, g)(
        u3, shaped(src_r, _r8(d3)), shaped(dst_r, _r8(d3)))
    return _tc_final(sp3, u3, dinv, b3, n)
